# baseline (speedup 1.0000x reference)
import functools

import jax
import jax.numpy as jnp
import numpy as np
from jax import lax
from jax.experimental import pallas as pl
from jax.experimental.pallas import tpu as pltpu
from jax.experimental.pallas import tpu_sc as plsc

VOCAB = 1000000
D_MODEL = 128
B = 4
L = 2048
PE_LEN = 2048
SCALE = float(np.sqrt(np.float64(D_MODEL)))

NUM_WORKERS = 32
ROWS_PER_W = (B * L) // NUM_WORKERS
CHUNK = 128
N_CHUNKS = ROWS_PER_W // CHUNK
LANES = 16


def _pe_table() -> np.ndarray:
    depth = D_MODEL / 2
    positions = np.arange(PE_LEN)[:, np.newaxis]
    depths = np.arange(depth)[np.newaxis, :] / depth
    angle_rads = positions * (1 / 10000**depths)
    return np.concatenate(
        [np.sin(angle_rads), np.cos(angle_rads)], axis=-1
    ).astype(np.float32)


_PE = _pe_table()


def _sc_body(x_hbm, pe_hbm, table_hbm, out_hbm,
             idx_v, acc_v, g_v, sem_g0, sem_g1, sem_pe, sem_st):
    wid = lax.axis_index("s") * 2 + lax.axis_index("c")
    b = wid // (L // ROWS_PER_W)
    l0 = lax.rem(wid, L // ROWS_PER_W) * ROWS_PER_W

    pltpu.sync_copy(x_hbm.at[b, pl.ds(l0, ROWS_PER_W)], idx_v)

    h_pe = pltpu.async_copy(pe_hbm.at[pl.ds(l0, ROWS_PER_W)], acc_v, sem_pe)
    h_g0 = pltpu.async_copy(
        table_hbm.at[idx_v.at[pl.ds(0, CHUNK)]],
        g_v.at[pl.ds(0, CHUNK)], sem_g0)
    h_g1 = pltpu.async_copy(
        table_hbm.at[idx_v.at[pl.ds(CHUNK, CHUNK)]],
        g_v.at[pl.ds(CHUNK, CHUNK)], sem_g1)

    UNROLL = 4

    def fma_rows(i, carry):
        for r in range(UNROLL):
            row = i * UNROLL + r
            for c in range(D_MODEL // LANES):
                sl = pl.ds(c * LANES, LANES)
                plsc.addupdate(acc_v.at[row, sl], g_v[row, sl] * SCALE)
        return carry

    h_pe.wait()
    h_g0.wait()
    lax.fori_loop(0, CHUNK // UNROLL, fma_rows, 0)
    h_st0 = pltpu.async_copy(
        acc_v.at[pl.ds(0, CHUNK)], out_hbm.at[b, pl.ds(l0, CHUNK)], sem_st)
    h_g1.wait()
    lax.fori_loop(CHUNK // UNROLL, ROWS_PER_W // UNROLL, fma_rows, 0)
    h_st0.wait()
    pltpu.sync_copy(
        acc_v.at[pl.ds(CHUNK, CHUNK)], out_hbm.at[b, pl.ds(l0 + CHUNK, CHUNK)])


def kernel(x, table):
    pe = jnp.asarray(_PE)

    sc_call = functools.partial(
        pl.kernel,
        out_type=jax.ShapeDtypeStruct((B, L, D_MODEL), jnp.float32),
        mesh=plsc.VectorSubcoreMesh(core_axis_name="c", subcore_axis_name="s"),
        scratch_types=[
            pltpu.VMEM((ROWS_PER_W,), jnp.int32),
            pltpu.VMEM((ROWS_PER_W, D_MODEL), jnp.float32),
            pltpu.VMEM((ROWS_PER_W, D_MODEL), jnp.float32),
            pltpu.SemaphoreType.DMA,
            pltpu.SemaphoreType.DMA,
            pltpu.SemaphoreType.DMA,
            pltpu.SemaphoreType.DMA,
        ],
    )(_sc_body)

    return sc_call(x, pe, table)


# speedup vs baseline: 1.0173x; 1.0173x over previous
import functools

import jax
import jax.numpy as jnp
import numpy as np
from jax import lax
from jax.experimental import pallas as pl
from jax.experimental.pallas import tpu as pltpu
from jax.experimental.pallas import tpu_sc as plsc

VOCAB = 1000000
D_MODEL = 128
B = 4
L = 2048
SCALE = float(np.sqrt(np.float64(D_MODEL)))

NUM_WORKERS = 32
ROWS_PER_W = (B * L) // NUM_WORKERS
CHUNK = 128
LANES = 16
HALF = D_MODEL // 2
NSEED = L // ROWS_PER_W
BLOCK = 64


def _const_table() -> np.ndarray:
    depths = np.arange(HALF, dtype=np.float64)[np.newaxis, :] / float(HALF)
    rates = 1.0 / 10000.0**depths
    seeds = np.zeros((NSEED + 2, D_MODEL), dtype=np.float64)
    pos = (np.arange(NSEED, dtype=np.float64) * ROWS_PER_W)[:, np.newaxis]
    seeds[:NSEED, :HALF] = np.sin(pos * rates)
    seeds[:NSEED, HALF:] = np.cos(pos * rates)
    seeds[NSEED, :HALF] = np.sin(rates)
    seeds[NSEED, HALF:] = np.sin(rates)
    seeds[NSEED + 1, :HALF] = np.cos(rates)
    seeds[NSEED + 1, HALF:] = np.cos(rates)
    return seeds.astype(np.float32)


_CONST = _const_table()
_NCHUNK = HALF // LANES


def _sc_body(x_hbm, cst_hbm, table_hbm, out_hbm, idx_v, g_v, cst_v,
             sem_g0, sem_g1, sem_st):
    wid = lax.axis_index("s") * 2 + lax.axis_index("c")
    b = wid // NSEED
    seed = lax.rem(wid, NSEED)
    l0 = seed * ROWS_PER_W

    pltpu.sync_copy(x_hbm.at[b, pl.ds(l0, ROWS_PER_W)], idx_v)
    pltpu.sync_copy(cst_hbm, cst_v)
    h_g0 = pltpu.async_copy(
        table_hbm.at[idx_v.at[pl.ds(0, CHUNK)]],
        g_v.at[pl.ds(0, CHUNK)], sem_g0)
    h_g1 = pltpu.async_copy(
        table_hbm.at[idx_v.at[pl.ds(CHUNK, CHUNK)]],
        g_v.at[pl.ds(CHUNK, CHUNK)], sem_g1)

    s = [cst_v[seed, pl.ds(j * LANES, LANES)] for j in range(_NCHUNK)]
    c = [cst_v[seed, pl.ds(HALF + j * LANES, LANES)] for j in range(_NCHUNK)]
    sr = [cst_v[NSEED, pl.ds(j * LANES, LANES)] for j in range(_NCHUNK)]
    cr = [cst_v[NSEED + 1, pl.ds(j * LANES, LANES)] for j in range(_NCHUNK)]

    def body(i, carry):
        sc = list(carry)
        for j in range(_NCHUNK):
            sl = pl.ds(j * LANES, LANES)
            g_v[i, sl] = g_v[i, sl] * SCALE + sc[j]
            slh = pl.ds(HALF + j * LANES, LANES)
            g_v[i, slh] = g_v[i, slh] * SCALE + sc[_NCHUNK + j]
        out = []
        for j in range(_NCHUNK):
            out.append(sc[j] * cr[j] + sc[_NCHUNK + j] * sr[j])
        for j in range(_NCHUNK):
            out.append(sc[_NCHUNK + j] * cr[j] - sc[j] * sr[j])
        return tuple(out)

    carry = tuple(s + c)
    h_store = None
    for blk in range(ROWS_PER_W // BLOCK):
        if blk * BLOCK == 0:
            h_g0.wait()
        if blk * BLOCK == CHUNK:
            h_g1.wait()
        carry = lax.fori_loop(blk * BLOCK, (blk + 1) * BLOCK, body, carry)
        if h_store is not None:
            h_store.wait()
        h_store = pltpu.async_copy(
            g_v.at[pl.ds(blk * BLOCK, BLOCK)],
            out_hbm.at[b, pl.ds(l0 + blk * BLOCK, BLOCK)], sem_st)
    h_store.wait()


def kernel(x, table):
    cst = jnp.asarray(_CONST)

    sc_call = functools.partial(
        pl.kernel,
        out_type=jax.ShapeDtypeStruct((B, L, D_MODEL), jnp.float32),
        mesh=plsc.VectorSubcoreMesh(core_axis_name="c", subcore_axis_name="s"),
        scratch_types=[
            pltpu.VMEM((ROWS_PER_W,), jnp.int32),
            pltpu.VMEM((ROWS_PER_W, D_MODEL), jnp.float32),
            pltpu.VMEM((NSEED + 2, D_MODEL), jnp.float32),
            pltpu.SemaphoreType.DMA,
            pltpu.SemaphoreType.DMA,
            pltpu.SemaphoreType.DMA,
        ],
    )(_sc_body)

    return sc_call(x, cst, table)


# speedup vs baseline: 1.0922x; 1.0736x over previous
import functools

import jax
import jax.numpy as jnp
import numpy as np
from jax import lax
from jax.experimental import pallas as pl
from jax.experimental.pallas import tpu as pltpu
from jax.experimental.pallas import tpu_sc as plsc

VOCAB = 1000000
D_MODEL = 128
B = 4
L = 2048
SCALE = float(np.sqrt(np.float64(D_MODEL)))
LN_1E4 = float(np.log(np.float64(10000.0)))

NUM_WORKERS = 32
POS_PER_W = L // NUM_WORKERS
HALF_POS = POS_PER_W // 2
LANES = 16
HALF = D_MODEL // 2
NCH = HALF // LANES
WID_BITS = 5


def _cmul(s1, c1, s2, c2):
    return s1 * c2 + c1 * s2, c1 * c2 - s1 * s2


def _sc_body(x_hbm, table_hbm, out_hbm, idx_v, g_v,
             sem_ix, sem_g0, sem_g1, sem_st):
    wid = lax.axis_index("s") * 2 + lax.axis_index("c")
    l0 = wid * POS_PER_W

    hi = [pltpu.async_copy(x_hbm.at[bb, pl.ds(l0, POS_PER_W)],
                           idx_v.at[pl.ds(bb * POS_PER_W, POS_PER_W)],
                           sem_ix)
          for bb in range(B)]
    for h in hi:
        h.wait()
    h0, h1 = [], []
    for bb in range(B):
        h0.append(pltpu.async_copy(
            table_hbm.at[idx_v.at[pl.ds(bb * POS_PER_W, HALF_POS)]],
            g_v.at[bb, pl.ds(0, HALF_POS)], sem_g0))
        h1.append(pltpu.async_copy(
            table_hbm.at[idx_v.at[pl.ds(bb * POS_PER_W + HALF_POS, HALF_POS)]],
            g_v.at[bb, pl.ds(HALF_POS, HALF_POS)], sem_g1))

    sr, cr = [], []
    for j in range(NCH):
        d = lax.iota(jnp.int32, LANES).astype(jnp.float32) + float(j * LANES)
        r = jnp.exp(d * (-LN_1E4 / HALF))
        x2 = r * r
        sp = 1.0 + x2 * (-1.0 / 6.0 + x2 * (1.0 / 120.0 + x2 * (
            -1.0 / 5040.0 + x2 * (1.0 / 362880.0))))
        cp = 1.0 + x2 * (-0.5 + x2 * (1.0 / 24.0 + x2 * (
            -1.0 / 720.0 + x2 * (1.0 / 40320.0 + x2 * (-1.0 / 3628800.0)))))
        sr.append(r * sp)
        cr.append(cp)

    p_s, p_c = list(sr), list(cr)
    for _ in range(6):
        for j in range(NCH):
            p_s[j], p_c[j] = _cmul(p_s[j], p_c[j], p_s[j], p_c[j])
    s = [jnp.zeros((LANES,), jnp.float32) for _ in range(NCH)]
    c = [jnp.ones((LANES,), jnp.float32) for _ in range(NCH)]
    for k in range(WID_BITS):
        bit = ((wid >> k) & 1).astype(jnp.float32)
        m = jnp.full((LANES,), 1.0, jnp.float32) * bit
        for j in range(NCH):
            ns, nc = _cmul(s[j], c[j], p_s[j], p_c[j])
            s[j] = s[j] + m * (ns - s[j])
            c[j] = c[j] + m * (nc - c[j])
        if k + 1 < WID_BITS:
            for j in range(NCH):
                p_s[j], p_c[j] = _cmul(p_s[j], p_c[j], p_s[j], p_c[j])

    def body(p, carry):
        sc = list(carry)
        for bb in range(B):
            for j in range(NCH):
                sl = pl.ds(j * LANES, LANES)
                g_v[bb, p, sl] = g_v[bb, p, sl] * SCALE + sc[j]
                slh = pl.ds(HALF + j * LANES, LANES)
                g_v[bb, p, slh] = g_v[bb, p, slh] * SCALE + sc[NCH + j]
        out = []
        for j in range(NCH):
            out.append(sc[j] * cr[j] + sc[NCH + j] * sr[j])
        for j in range(NCH):
            out.append(sc[NCH + j] * cr[j] - sc[j] * sr[j])
        return tuple(out)

    carry = tuple(s + c)
    for h in h0:
        h.wait()
    carry = lax.fori_loop(0, HALF_POS, body, carry)
    st0 = [pltpu.async_copy(
        g_v.at[bb, pl.ds(0, HALF_POS)],
        out_hbm.at[bb, pl.ds(l0, HALF_POS)], sem_st) for bb in range(B)]
    for h in h1:
        h.wait()
    lax.fori_loop(HALF_POS, POS_PER_W, body, carry)
    st1 = [pltpu.async_copy(
        g_v.at[bb, pl.ds(HALF_POS, HALF_POS)],
        out_hbm.at[bb, pl.ds(l0 + HALF_POS, HALF_POS)], sem_st)
        for bb in range(B)]
    for h in st0 + st1:
        h.wait()


def kernel(x, table):
    sc_call = functools.partial(
        pl.kernel,
        out_type=jax.ShapeDtypeStruct((B, L, D_MODEL), jnp.float32),
        mesh=plsc.VectorSubcoreMesh(core_axis_name="c", subcore_axis_name="s"),
        scratch_types=[
            pltpu.VMEM((B * POS_PER_W,), jnp.int32),
            pltpu.VMEM((B, POS_PER_W, D_MODEL), jnp.float32),
            pltpu.SemaphoreType.DMA,
            pltpu.SemaphoreType.DMA,
            pltpu.SemaphoreType.DMA,
            pltpu.SemaphoreType.DMA,
        ],
    )(_sc_body)

    return sc_call(x, table)
